# Initial kernel scaffold; baseline (speedup 1.0000x reference)
import jax
import jax.numpy as jnp
from jax.experimental import pallas as pl


def kernel(x, edge_index, edge_attr, batch, We0, be0, W10, b10, W20, b20, gw0, gb0, gm0, We1, be1, W11, b11, W21, b21, gw1, gb1, gm1, Wl, bl):
    raise NotImplementedError("write your pallas kernel here")


# speedup vs baseline: 2.4698x; 2.4698x over previous
import functools

import jax
import jax.numpy as jnp
from jax import lax
from jax.experimental import pallas as pl
from jax.experimental.pallas import tpu as pltpu
from jax.experimental.pallas import tpu_sc as plsc

N = 10000
E = 320000
D = 128
G = 128

NP = 10240
TRASH = N
NW = 32
CH = 128
EP = ((E + NW * CH - 1) // (NW * CH)) * (NW * CH)
CPW = EP // (NW * CH)
NB = 2560
EB = 4096

_HI = jax.lax.Precision.HIGHEST


def _dot(a, b):
    return jnp.dot(a, b, precision=_HI, preferred_element_type=jnp.float32)


def _dot_t(a, b):
    return lax.dot_general(a, b, (((0,), (0,)), ((), ())), precision=_HI,
                           preferred_element_type=jnp.float32)


def _edge_embed_body(ea_ref, we_ref, be_ref, out_ref):
    out_ref[...] = _dot(ea_ref[...], we_ref[...]) + be_ref[...]


def _edge_embed(ea_p, We, be):
    return pl.pallas_call(
        _edge_embed_body,
        grid=(EP // EB,),
        in_specs=[
            pl.BlockSpec((EB, 16), lambda i: (i, 0)),
            pl.BlockSpec((16, D), lambda i: (0, 0)),
            pl.BlockSpec((1, D), lambda i: (0, 0)),
        ],
        out_specs=pl.BlockSpec((EB, D), lambda i: (i, 0)),
        out_shape=jax.ShapeDtypeStruct((EP, D), jnp.float32),
    )(ea_p, We, be.reshape(1, D))


def _sc_agg_body(src_hbm, dst_hbm, eemb_hbm, h_hbm, zeros_hbm, out_hbm,
                 src_v, dst_v, m_v, e_v, agg_sh, sem):
    c = lax.axis_index("c")
    s = lax.axis_index("s")
    wid = s * 2 + c
    rpt = NP // 16
    pltpu.sync_copy(zeros_hbm.at[pl.ds(s * rpt, rpt)],
                    agg_sh.at[pl.ds(s * rpt, rpt)])
    plsc.subcore_barrier()

    @pl.loop(0, CPW)
    def _(i):
        base = (i * NW + wid) * CH
        pltpu.sync_copy(src_hbm.at[pl.ds(base, CH)], src_v)
        pltpu.sync_copy(dst_hbm.at[pl.ds(base, CH)], dst_v)
        pltpu.async_copy(h_hbm.at[src_v], m_v, sem).wait()
        pltpu.sync_copy(eemb_hbm.at[pl.ds(base, CH)], e_v)

        @pl.loop(0, CH)
        def _(r):
            for j in range(D // 16):
                sl = (r, pl.ds(j * 16, 16))
                m_v.at[*sl][...] = jnp.maximum(
                    m_v.at[*sl][...] + e_v.at[*sl][...], 0.0)

        pltpu.sync_copy(m_v, agg_sh.at[dst_v], add=True)

    plsc.subcore_barrier()
    pltpu.sync_copy(agg_sh.at[pl.ds(s * rpt, rpt)],
                    out_hbm.at[c, pl.ds(s * rpt, rpt)])


@functools.lru_cache(maxsize=1)
def _make_sc_agg():
    mesh = plsc.VectorSubcoreMesh(core_axis_name="c", subcore_axis_name="s",
                                  num_cores=2, num_subcores=16)
    return pl.kernel(
        _sc_agg_body,
        out_type=jax.ShapeDtypeStruct((2, NP, D), jnp.float32),
        mesh=mesh,
        scratch_types=[
            pltpu.VMEM((CH,), jnp.int32),
            pltpu.VMEM((CH,), jnp.int32),
            pltpu.VMEM((CH, D), jnp.float32),
            pltpu.VMEM((CH, D), jnp.float32),
            pltpu.VMEM_SHARED((NP, D), jnp.float32),
            pltpu.SemaphoreType.DMA,
        ],
    )


def _sc_agg(src_p, dst_p, eemb, h, zeros_nd):
    return _make_sc_agg()(src_p, dst_p, eemb, h, zeros_nd)


def _mlp_stats_body(h_ref, agg_ref, b_ref, w1_ref, b1_ref, w2_ref, b2_ref,
                    z2_ref, segz_ref, segz2_ref, cnt_ref):
    i = pl.program_id(0)
    z = h_ref[...] + agg_ref[0] + agg_ref[1]
    z1 = jnp.maximum(_dot(z, w1_ref[...]) + b1_ref[...], 0.0)
    z2 = _dot(z1, w2_ref[...]) + b2_ref[...]
    z2_ref[...] = z2
    gids = lax.broadcasted_iota(jnp.int32, (1, G), 1)
    oneh = (b_ref[...] == gids).astype(jnp.float32)

    @pl.when(i == 0)
    def _():
        segz_ref[...] = jnp.zeros_like(segz_ref)
        segz2_ref[...] = jnp.zeros_like(segz2_ref)
        cnt_ref[...] = jnp.zeros_like(cnt_ref)

    segz_ref[...] += _dot_t(oneh, z2)
    segz2_ref[...] += _dot_t(oneh, z2 * z2)
    cnt_ref[...] += _dot_t(oneh, jnp.ones((NB, 1), jnp.float32))


def _mlp_stats(h, aggpair, batch2d, W1, b1, W2, b2):
    return pl.pallas_call(
        _mlp_stats_body,
        grid=(NP // NB,),
        in_specs=[
            pl.BlockSpec((NB, D), lambda i: (i, 0)),
            pl.BlockSpec((2, NB, D), lambda i: (0, i, 0)),
            pl.BlockSpec((NB, 1), lambda i: (i, 0)),
            pl.BlockSpec((D, D), lambda i: (0, 0)),
            pl.BlockSpec((1, D), lambda i: (0, 0)),
            pl.BlockSpec((D, D), lambda i: (0, 0)),
            pl.BlockSpec((1, D), lambda i: (0, 0)),
        ],
        out_specs=[
            pl.BlockSpec((NB, D), lambda i: (i, 0)),
            pl.BlockSpec((G, D), lambda i: (0, 0)),
            pl.BlockSpec((G, D), lambda i: (0, 0)),
            pl.BlockSpec((G, 1), lambda i: (0, 0)),
        ],
        out_shape=[
            jax.ShapeDtypeStruct((NP, D), jnp.float32),
            jax.ShapeDtypeStruct((G, D), jnp.float32),
            jax.ShapeDtypeStruct((G, D), jnp.float32),
            jax.ShapeDtypeStruct((G, 1), jnp.float32),
        ],
    )(h, aggpair, batch2d, W1, b1.reshape(1, D), W2, b2.reshape(1, D))


def _norm_body(z2_ref, b_ref, segz_ref, segz2_ref, cnt_ref,
               gw_ref, gb_ref, gm_ref, out_ref, pool_ref):
    i = pl.program_id(0)
    cnt = jnp.maximum(cnt_ref[...], 1.0)
    mean = segz_ref[...] / cnt
    msq = segz2_ref[...] / cnt
    gm = gm_ref[...]
    var = msq - (2.0 - gm) * gm * mean * mean
    inv = gw_ref[...] * lax.rsqrt(var + 1e-5)
    alpha = inv
    beta = gb_ref[...] - inv * gm * mean
    gids = lax.broadcasted_iota(jnp.int32, (1, G), 1)
    oneh = (b_ref[...] == gids).astype(jnp.float32)
    a_b = _dot(oneh, alpha)
    b_b = _dot(oneh, beta)
    out = jnp.maximum(a_b * z2_ref[...] + b_b, 0.0)
    out_ref[...] = out

    @pl.when(i == 0)
    def _():
        pool_ref[...] = jnp.zeros_like(pool_ref)

    pool_ref[...] += _dot_t(oneh, out)


def _norm(z2, batch2d, segz, segz2, cnt, gw, gb, gm):
    return pl.pallas_call(
        _norm_body,
        grid=(NP // NB,),
        in_specs=[
            pl.BlockSpec((NB, D), lambda i: (i, 0)),
            pl.BlockSpec((NB, 1), lambda i: (i, 0)),
            pl.BlockSpec((G, D), lambda i: (0, 0)),
            pl.BlockSpec((G, D), lambda i: (0, 0)),
            pl.BlockSpec((G, 1), lambda i: (0, 0)),
            pl.BlockSpec((1, D), lambda i: (0, 0)),
            pl.BlockSpec((1, D), lambda i: (0, 0)),
            pl.BlockSpec((1, D), lambda i: (0, 0)),
        ],
        out_specs=[
            pl.BlockSpec((NB, D), lambda i: (i, 0)),
            pl.BlockSpec((G, D), lambda i: (0, 0)),
        ],
        out_shape=[
            jax.ShapeDtypeStruct((NP, D), jnp.float32),
            jax.ShapeDtypeStruct((G, D), jnp.float32),
        ],
    )(z2, batch2d, segz, segz2, cnt,
      gw.reshape(1, D), gb.reshape(1, D), gm.reshape(1, D))


def _final_body(pool_ref, cnt_ref, wl_ref, bl_ref, out_ref):
    cnt = jnp.maximum(cnt_ref[...], 1.0)
    pooled = pool_ref[...] / cnt
    out_ref[...] = _dot(pooled, wl_ref[...]) + bl_ref[...]


def _final(pool, cnt, Wl, bl):
    return pl.pallas_call(
        _final_body,
        out_shape=jax.ShapeDtypeStruct((G, D), jnp.float32),
    )(pool, cnt, Wl, bl.reshape(1, D))


def kernel(x, edge_index, edge_attr, batch,
           We0, be0, W10, b10, W20, b20, gw0, gb0, gm0,
           We1, be1, W11, b11, W21, b21, gw1, gb1, gm1, Wl, bl):
    src = edge_index[0].astype(jnp.int32)
    dst = edge_index[1].astype(jnp.int32)
    src_p = jnp.concatenate([src, jnp.zeros((EP - E,), jnp.int32)])
    dst_p = jnp.concatenate([dst, jnp.full((EP - E,), TRASH, jnp.int32)])
    ea_p = jnp.concatenate(
        [edge_attr, jnp.zeros((EP - E, 16), jnp.float32)], axis=0)
    x_p = jnp.concatenate([x, jnp.zeros((NP - N, D), jnp.float32)], axis=0)
    batch2d = jnp.concatenate(
        [batch.astype(jnp.int32), jnp.full((NP - N,), G, jnp.int32)]
    ).reshape(NP, 1)
    zeros_nd = jnp.zeros((NP, D), jnp.float32)

    eemb0 = _edge_embed(ea_p, We0, be0)
    eemb1 = _edge_embed(ea_p, We1, be1)

    agg0 = _sc_agg(src_p, dst_p, eemb0, x_p, zeros_nd)
    z2_0, segz0, segz20, cnt = _mlp_stats(x_p, agg0, batch2d, W10, b10, W20, b20)
    h1, _ = _norm(z2_0, batch2d, segz0, segz20, cnt, gw0, gb0, gm0)

    agg1 = _sc_agg(src_p, dst_p, eemb1, h1, zeros_nd)
    z2_1, segz1, segz21, _ = _mlp_stats(h1, agg1, batch2d, W11, b11, W21, b21)
    _, pool = _norm(z2_1, batch2d, segz1, segz21, cnt, gw1, gb1, gm1)

    return _final(pool, cnt, Wl, bl)


# speedup vs baseline: 3.4791x; 1.4087x over previous
import functools

import jax
import jax.numpy as jnp
from jax import lax
from jax.experimental import pallas as pl
from jax.experimental.pallas import tpu as pltpu
from jax.experimental.pallas import tpu_sc as plsc

N = 10000
E = 320000
D = 128
G = 128

NP = 10240
TRASH = N
NW = 32
CH = 80
NBUF = 2
CPW = ((E + NW * CH - 1) // (NW * CH) + NBUF - 1) // NBUF * NBUF
EP = NW * CH * CPW
NB = 2560
EB = 2560

_HI = jax.lax.Precision.HIGHEST


def _dot(a, b):
    return jnp.dot(a, b, precision=_HI, preferred_element_type=jnp.float32)


def _dot_t(a, b):
    return lax.dot_general(a, b, (((0,), (0,)), ((), ())), precision=_HI,
                           preferred_element_type=jnp.float32)


def _edge_embed_body(ea_ref, we_ref, be_ref, out_ref):
    out_ref[...] = _dot(ea_ref[...], we_ref[...]) + be_ref[...]


def _edge_embed(ea_p, We, be):
    return pl.pallas_call(
        _edge_embed_body,
        grid=(EP // EB,),
        in_specs=[
            pl.BlockSpec((EB, 16), lambda i: (i, 0)),
            pl.BlockSpec((16, D), lambda i: (0, 0)),
            pl.BlockSpec((1, D), lambda i: (0, 0)),
        ],
        out_specs=pl.BlockSpec((EB, D), lambda i: (i, 0)),
        out_shape=jax.ShapeDtypeStruct((EP, D), jnp.float32),
    )(ea_p, We, be.reshape(1, D))


def _sc_agg_body(src_hbm, dst_hbm, eemb_hbm, h_hbm, zeros_hbm, out_hbm,
                 sidx0, sidx1, didx0, didx1, m_bufs, e_bufs, agg_sh,
                 isem, gsem):
    sidx = (sidx0, sidx1)
    didx = (didx0, didx1)
    c = lax.axis_index("c")
    s = lax.axis_index("s")
    wid = s * 2 + c
    rpt = NP // 16
    pltpu.sync_copy(zeros_hbm.at[pl.ds(s * rpt, rpt)],
                    agg_sh.at[pl.ds(s * rpt, rpt)])

    def issue_idx(k, b):
        pltpu.async_copy(src_hbm.at[wid, k], sidx[b], isem.at[b])
        pltpu.async_copy(dst_hbm.at[wid, k], didx[b], isem.at[b])

    def wait_idx(k, b):
        pltpu.make_async_copy(src_hbm.at[wid, k], sidx[b],
                              isem.at[b]).wait()
        pltpu.make_async_copy(dst_hbm.at[wid, k], didx[b],
                              isem.at[b]).wait()

    def issue_gather(k, b):
        base = (wid * CPW + k) * CH
        pltpu.async_copy(h_hbm.at[sidx[b]], m_bufs.at[b], gsem.at[b])
        pltpu.async_copy(eemb_hbm.at[pl.ds(base, CH)], e_bufs.at[b],
                         gsem.at[b])

    def wait_gather(k, b):
        base = (wid * CPW + k) * CH
        pltpu.make_async_copy(h_hbm.at[sidx[b]], m_bufs.at[b],
                              gsem.at[b]).wait()
        pltpu.make_async_copy(eemb_hbm.at[pl.ds(base, CH)], e_bufs.at[b],
                              gsem.at[b]).wait()

    issue_idx(0, 0)
    issue_idx(1, 1)
    wait_idx(0, 0)
    issue_gather(0, 0)
    plsc.subcore_barrier()

    @pl.loop(0, CPW, step=NBUF)
    def _(i):
        for bb in range(NBUF):
            k = i + bb
            ob = 1 - bb

            @pl.when(k + 1 < CPW)
            def _():
                wait_idx(k + 1, ob)
                issue_gather(k + 1, ob)

            wait_gather(k, bb)
            m_v = m_bufs.at[bb]
            e_v = e_bufs.at[bb]

            @pl.loop(0, CH)
            def _(r):
                for j in range(D // 16):
                    sl = (r, pl.ds(j * 16, 16))
                    m_v.at[*sl][...] = jnp.maximum(
                        m_v.at[*sl][...] + e_v.at[*sl][...], 0.0)

            pltpu.sync_copy(m_v, agg_sh.at[didx[bb]], add=True)

            @pl.when(k + 2 < CPW)
            def _():
                issue_idx(k + 2, bb)

    plsc.subcore_barrier()
    pltpu.sync_copy(agg_sh.at[pl.ds(s * rpt, rpt)],
                    out_hbm.at[c, pl.ds(s * rpt, rpt)])


@functools.lru_cache(maxsize=1)
def _make_sc_agg():
    mesh = plsc.VectorSubcoreMesh(core_axis_name="c", subcore_axis_name="s",
                                  num_cores=2, num_subcores=16)
    return pl.kernel(
        _sc_agg_body,
        out_type=jax.ShapeDtypeStruct((2, NP, D), jnp.float32),
        mesh=mesh,
        scratch_types=[
            pltpu.VMEM((CH,), jnp.int32),
            pltpu.VMEM((CH,), jnp.int32),
            pltpu.VMEM((CH,), jnp.int32),
            pltpu.VMEM((CH,), jnp.int32),
            pltpu.VMEM((NBUF, CH, D), jnp.float32),
            pltpu.VMEM((NBUF, CH, D), jnp.float32),
            pltpu.VMEM_SHARED((NP, D), jnp.float32),
            pltpu.SemaphoreType.DMA((NBUF,)),
            pltpu.SemaphoreType.DMA((NBUF,)),
        ],
    )


def _sc_agg(src2d, dst2d, eemb, h, zeros_nd):
    return _make_sc_agg()(src2d, dst2d, eemb, h, zeros_nd)


def _mlp_stats_body(h_ref, agg_ref, b_ref, w1_ref, b1_ref, w2_ref, b2_ref,
                    z2_ref, segz_ref, segz2_ref, cnt_ref):
    i = pl.program_id(0)
    z = h_ref[...] + agg_ref[0] + agg_ref[1]
    z1 = jnp.maximum(_dot(z, w1_ref[...]) + b1_ref[...], 0.0)
    z2 = _dot(z1, w2_ref[...]) + b2_ref[...]
    z2_ref[...] = z2
    gids = lax.broadcasted_iota(jnp.int32, (1, G), 1)
    oneh = (b_ref[...] == gids).astype(jnp.float32)

    @pl.when(i == 0)
    def _():
        segz_ref[...] = jnp.zeros_like(segz_ref)
        segz2_ref[...] = jnp.zeros_like(segz2_ref)
        cnt_ref[...] = jnp.zeros_like(cnt_ref)

    segz_ref[...] += _dot_t(oneh, z2)
    segz2_ref[...] += _dot_t(oneh, z2 * z2)
    cnt_ref[...] += _dot_t(oneh, jnp.ones((NB, 1), jnp.float32))


def _mlp_stats(h, aggpair, batch2d, W1, b1, W2, b2):
    return pl.pallas_call(
        _mlp_stats_body,
        grid=(NP // NB,),
        in_specs=[
            pl.BlockSpec((NB, D), lambda i: (i, 0)),
            pl.BlockSpec((2, NB, D), lambda i: (0, i, 0)),
            pl.BlockSpec((NB, 1), lambda i: (i, 0)),
            pl.BlockSpec((D, D), lambda i: (0, 0)),
            pl.BlockSpec((1, D), lambda i: (0, 0)),
            pl.BlockSpec((D, D), lambda i: (0, 0)),
            pl.BlockSpec((1, D), lambda i: (0, 0)),
        ],
        out_specs=[
            pl.BlockSpec((NB, D), lambda i: (i, 0)),
            pl.BlockSpec((G, D), lambda i: (0, 0)),
            pl.BlockSpec((G, D), lambda i: (0, 0)),
            pl.BlockSpec((G, 1), lambda i: (0, 0)),
        ],
        out_shape=[
            jax.ShapeDtypeStruct((NP, D), jnp.float32),
            jax.ShapeDtypeStruct((G, D), jnp.float32),
            jax.ShapeDtypeStruct((G, D), jnp.float32),
            jax.ShapeDtypeStruct((G, 1), jnp.float32),
        ],
    )(h, aggpair, batch2d, W1, b1.reshape(1, D), W2, b2.reshape(1, D))


def _norm_body(z2_ref, b_ref, segz_ref, segz2_ref, cnt_ref,
               gw_ref, gb_ref, gm_ref, out_ref, pool_ref):
    i = pl.program_id(0)
    cnt = jnp.maximum(cnt_ref[...], 1.0)
    mean = segz_ref[...] / cnt
    msq = segz2_ref[...] / cnt
    gm = gm_ref[...]
    var = msq - (2.0 - gm) * gm * mean * mean
    inv = gw_ref[...] * lax.rsqrt(var + 1e-5)
    alpha = inv
    beta = gb_ref[...] - inv * gm * mean
    gids = lax.broadcasted_iota(jnp.int32, (1, G), 1)
    oneh = (b_ref[...] == gids).astype(jnp.float32)
    a_b = _dot(oneh, alpha)
    b_b = _dot(oneh, beta)
    out = jnp.maximum(a_b * z2_ref[...] + b_b, 0.0)
    out_ref[...] = out

    @pl.when(i == 0)
    def _():
        pool_ref[...] = jnp.zeros_like(pool_ref)

    pool_ref[...] += _dot_t(oneh, out)


def _norm(z2, batch2d, segz, segz2, cnt, gw, gb, gm):
    return pl.pallas_call(
        _norm_body,
        grid=(NP // NB,),
        in_specs=[
            pl.BlockSpec((NB, D), lambda i: (i, 0)),
            pl.BlockSpec((NB, 1), lambda i: (i, 0)),
            pl.BlockSpec((G, D), lambda i: (0, 0)),
            pl.BlockSpec((G, D), lambda i: (0, 0)),
            pl.BlockSpec((G, 1), lambda i: (0, 0)),
            pl.BlockSpec((1, D), lambda i: (0, 0)),
            pl.BlockSpec((1, D), lambda i: (0, 0)),
            pl.BlockSpec((1, D), lambda i: (0, 0)),
        ],
        out_specs=[
            pl.BlockSpec((NB, D), lambda i: (i, 0)),
            pl.BlockSpec((G, D), lambda i: (0, 0)),
        ],
        out_shape=[
            jax.ShapeDtypeStruct((NP, D), jnp.float32),
            jax.ShapeDtypeStruct((G, D), jnp.float32),
        ],
    )(z2, batch2d, segz, segz2, cnt,
      gw.reshape(1, D), gb.reshape(1, D), gm.reshape(1, D))


def _final_body(pool_ref, cnt_ref, wl_ref, bl_ref, out_ref):
    cnt = jnp.maximum(cnt_ref[...], 1.0)
    pooled = pool_ref[...] / cnt
    out_ref[...] = _dot(pooled, wl_ref[...]) + bl_ref[...]


def _final(pool, cnt, Wl, bl):
    return pl.pallas_call(
        _final_body,
        out_shape=jax.ShapeDtypeStruct((G, D), jnp.float32),
    )(pool, cnt, Wl, bl.reshape(1, D))


def kernel(x, edge_index, edge_attr, batch,
           We0, be0, W10, b10, W20, b20, gw0, gb0, gm0,
           We1, be1, W11, b11, W21, b21, gw1, gb1, gm1, Wl, bl):
    src = edge_index[0].astype(jnp.int32)
    dst = edge_index[1].astype(jnp.int32)
    src_p = jnp.concatenate(
        [src, jnp.zeros((EP - E,), jnp.int32)]).reshape(NW, CPW, CH)
    dst_p = jnp.concatenate(
        [dst, jnp.full((EP - E,), TRASH, jnp.int32)]).reshape(NW, CPW, CH)
    ea_p = jnp.concatenate(
        [edge_attr, jnp.zeros((EP - E, 16), jnp.float32)], axis=0)
    x_p = jnp.concatenate([x, jnp.zeros((NP - N, D), jnp.float32)], axis=0)
    batch2d = jnp.concatenate(
        [batch.astype(jnp.int32), jnp.full((NP - N,), G, jnp.int32)]
    ).reshape(NP, 1)
    zeros_nd = jnp.zeros((NP, D), jnp.float32)

    eemb0 = _edge_embed(ea_p, We0, be0)
    eemb1 = _edge_embed(ea_p, We1, be1)

    agg0 = _sc_agg(src_p, dst_p, eemb0, x_p, zeros_nd)
    z2_0, segz0, segz20, cnt = _mlp_stats(x_p, agg0, batch2d, W10, b10, W20, b20)
    h1, _ = _norm(z2_0, batch2d, segz0, segz20, cnt, gw0, gb0, gm0)

    agg1 = _sc_agg(src_p, dst_p, eemb1, h1, zeros_nd)
    z2_1, segz1, segz21, _ = _mlp_stats(h1, agg1, batch2d, W11, b11, W21, b21)
    _, pool = _norm(z2_1, batch2d, segz1, segz21, cnt, gw1, gb1, gm1)

    return _final(pool, cnt, Wl, bl)


# speedup vs baseline: 3.7180x; 1.0686x over previous
import functools

import jax
import jax.numpy as jnp
from jax import lax
from jax.experimental import pallas as pl
from jax.experimental.pallas import tpu as pltpu
from jax.experimental.pallas import tpu_sc as plsc

N = 10000
E = 320000
D = 128
G = 128

NP = 10240
TRASH = N
NW = 32
CH = 80
NBUF = 2
CPW = ((E + NW * CH - 1) // (NW * CH) + 5) // 6 * 6
EP = NW * CH * CPW
NB = 2560
EB = 8064

_HI = jax.lax.Precision.HIGHEST


def _dot(a, b, precision=_HI):
    return jnp.dot(a, b, precision=precision,
                   preferred_element_type=jnp.float32)


def _dot_t(a, b):
    return lax.dot_general(a, b, (((0,), (0,)), ((), ())), precision=_HI,
                           preferred_element_type=jnp.float32)


def _edge_embed_body(ea_ref, we_ref, be_ref, out_ref):
    out_ref[...] = _dot(ea_ref[...], we_ref[...]) + be_ref[...]


def _edge_embed(ea_p, We, be):
    return pl.pallas_call(
        _edge_embed_body,
        grid=(EP // EB,),
        in_specs=[
            pl.BlockSpec((EB, 16), lambda i: (i, 0)),
            pl.BlockSpec((16, D), lambda i: (0, 0)),
            pl.BlockSpec((1, D), lambda i: (0, 0)),
        ],
        out_specs=pl.BlockSpec((EB, D), lambda i: (i, 0)),
        out_shape=jax.ShapeDtypeStruct((EP, D), jnp.float32),
    )(ea_p, We, be.reshape(1, D))


def _sc_agg_body(src_hbm, dst_hbm, eemb_hbm, h_hbm, zeros_hbm, out_hbm,
                 sidx0, sidx1, didx0, didx1, didx2, m_bufs, e_bufs, agg_sh,
                 isem, gsem, ssem):
    sidx = (sidx0, sidx1)
    didx = (didx0, didx1, didx2)
    c = lax.axis_index("c")
    s = lax.axis_index("s")
    wid = s * 2 + c
    rpt = NP // 16
    pltpu.sync_copy(zeros_hbm.at[pl.ds(s * rpt, rpt)],
                    agg_sh.at[pl.ds(s * rpt, rpt)])

    def issue_idx(k, b, d):
        pltpu.async_copy(src_hbm.at[wid, k], sidx[b], isem.at[b])
        pltpu.async_copy(dst_hbm.at[wid, k], didx[d], isem.at[b])

    def wait_idx(k, b, d):
        pltpu.make_async_copy(src_hbm.at[wid, k], sidx[b],
                              isem.at[b]).wait()
        pltpu.make_async_copy(dst_hbm.at[wid, k], didx[d],
                              isem.at[b]).wait()

    def issue_gather(k, b):
        base = (wid * CPW + k) * CH
        pltpu.async_copy(h_hbm.at[sidx[b]], m_bufs.at[b], gsem.at[b])
        pltpu.async_copy(eemb_hbm.at[pl.ds(base, CH)], e_bufs.at[b],
                         gsem.at[b])

    def wait_gather(k, b):
        base = (wid * CPW + k) * CH
        pltpu.make_async_copy(h_hbm.at[sidx[b]], m_bufs.at[b],
                              gsem.at[b]).wait()
        pltpu.make_async_copy(eemb_hbm.at[pl.ds(base, CH)], e_bufs.at[b],
                              gsem.at[b]).wait()

    def wait_scatter(b, d):
        pltpu.make_async_copy(m_bufs.at[b], agg_sh.at[didx[d]],
                              ssem.at[b]).wait()

    issue_idx(0, 0, 0)
    issue_idx(1, 1, 1)
    wait_idx(0, 0, 0)
    issue_gather(0, 0)
    plsc.subcore_barrier()

    @pl.loop(0, CPW, step=6)
    def _(i):
        for bb in range(6):
            k = i + bb
            b = bb % NBUF
            ob = 1 - b
            d = bb % 3
            d1 = (bb + 1) % 3
            d2 = (bb + 2) % 3

            @pl.when(k >= 1)
            def _():
                wait_scatter(ob, d2)

            @pl.when(k + 1 < CPW)
            def _():
                wait_idx(k + 1, ob, d1)
                issue_gather(k + 1, ob)

            wait_gather(k, b)
            m_v = m_bufs.at[b]
            e_v = e_bufs.at[b]

            @pl.loop(0, CH, step=2)
            def _(r):
                for rr in range(2):
                    for j in range(D // 16):
                        sl = (r + rr, pl.ds(j * 16, 16))
                        m_v.at[*sl][...] = jnp.maximum(
                            m_v.at[*sl][...] + e_v.at[*sl][...], 0.0)

            pltpu.async_copy(m_v, agg_sh.at[didx[d]], ssem.at[b], add=True)

            @pl.when(k + 2 < CPW)
            def _():
                issue_idx(k + 2, b, d2)

    wait_scatter((CPW - 1) % 2, (CPW - 1) % 3)
    plsc.subcore_barrier()
    pltpu.sync_copy(agg_sh.at[pl.ds(s * rpt, rpt)],
                    out_hbm.at[c, pl.ds(s * rpt, rpt)])


@functools.lru_cache(maxsize=1)
def _make_sc_agg():
    mesh = plsc.VectorSubcoreMesh(core_axis_name="c", subcore_axis_name="s",
                                  num_cores=2, num_subcores=16)
    return pl.kernel(
        _sc_agg_body,
        out_type=jax.ShapeDtypeStruct((2, NP, D), jnp.float32),
        mesh=mesh,
        scratch_types=[
            pltpu.VMEM((CH,), jnp.int32),
            pltpu.VMEM((CH,), jnp.int32),
            pltpu.VMEM((CH,), jnp.int32),
            pltpu.VMEM((CH,), jnp.int32),
            pltpu.VMEM((CH,), jnp.int32),
            pltpu.VMEM((NBUF, CH, D), jnp.float32),
            pltpu.VMEM((NBUF, CH, D), jnp.float32),
            pltpu.VMEM_SHARED((NP, D), jnp.float32),
            pltpu.SemaphoreType.DMA((NBUF,)),
            pltpu.SemaphoreType.DMA((NBUF,)),
            pltpu.SemaphoreType.DMA((NBUF,)),
        ],
    )


def _sc_agg(src2d, dst2d, eemb, h, zeros_nd):
    return _make_sc_agg()(src2d, dst2d, eemb, h, zeros_nd)


def _mlp_stats_body(h_ref, agg_ref, b_ref, w1_ref, b1_ref, w2_ref, b2_ref,
                    z2_ref, segz_ref, segz2_ref, cnt_ref):
    i = pl.program_id(0)
    z = h_ref[...] + agg_ref[0] + agg_ref[1]
    z1 = jnp.maximum(_dot(z, w1_ref[...]) + b1_ref[...], 0.0)
    z2 = _dot(z1, w2_ref[...]) + b2_ref[...]
    z2_ref[...] = z2
    gids = lax.broadcasted_iota(jnp.int32, (1, G), 1)
    oneh = (b_ref[...] == gids).astype(jnp.float32)

    @pl.when(i == 0)
    def _():
        segz_ref[...] = jnp.zeros_like(segz_ref)
        segz2_ref[...] = jnp.zeros_like(segz2_ref)
        cnt_ref[...] = jnp.zeros_like(cnt_ref)

    segz_ref[...] += _dot_t(oneh, z2)
    segz2_ref[...] += _dot_t(oneh, z2 * z2)
    cnt_ref[...] += _dot_t(oneh, jnp.ones((NB, 1), jnp.float32))


def _mlp_stats(h, aggpair, batch2d, W1, b1, W2, b2):
    return pl.pallas_call(
        _mlp_stats_body,
        grid=(NP // NB,),
        in_specs=[
            pl.BlockSpec((NB, D), lambda i: (i, 0)),
            pl.BlockSpec((2, NB, D), lambda i: (0, i, 0)),
            pl.BlockSpec((NB, 1), lambda i: (i, 0)),
            pl.BlockSpec((D, D), lambda i: (0, 0)),
            pl.BlockSpec((1, D), lambda i: (0, 0)),
            pl.BlockSpec((D, D), lambda i: (0, 0)),
            pl.BlockSpec((1, D), lambda i: (0, 0)),
        ],
        out_specs=[
            pl.BlockSpec((NB, D), lambda i: (i, 0)),
            pl.BlockSpec((G, D), lambda i: (0, 0)),
            pl.BlockSpec((G, D), lambda i: (0, 0)),
            pl.BlockSpec((G, 1), lambda i: (0, 0)),
        ],
        out_shape=[
            jax.ShapeDtypeStruct((NP, D), jnp.float32),
            jax.ShapeDtypeStruct((G, D), jnp.float32),
            jax.ShapeDtypeStruct((G, D), jnp.float32),
            jax.ShapeDtypeStruct((G, 1), jnp.float32),
        ],
    )(h, aggpair, batch2d, W1, b1.reshape(1, D), W2, b2.reshape(1, D))


def _norm_body(z2_ref, b_ref, segz_ref, segz2_ref, cnt_ref,
               gw_ref, gb_ref, gm_ref, out_ref, pool_ref):
    i = pl.program_id(0)
    cnt = jnp.maximum(cnt_ref[...], 1.0)
    mean = segz_ref[...] / cnt
    msq = segz2_ref[...] / cnt
    gm = gm_ref[...]
    var = msq - (2.0 - gm) * gm * mean * mean
    inv = gw_ref[...] * lax.rsqrt(var + 1e-5)
    alpha = inv
    beta = gb_ref[...] - inv * gm * mean
    gids = lax.broadcasted_iota(jnp.int32, (1, G), 1)
    oneh = (b_ref[...] == gids).astype(jnp.float32)
    a_b = _dot(oneh, alpha)
    b_b = _dot(oneh, beta)
    out = jnp.maximum(a_b * z2_ref[...] + b_b, 0.0)
    out_ref[...] = out

    @pl.when(i == 0)
    def _():
        pool_ref[...] = jnp.zeros_like(pool_ref)

    pool_ref[...] += _dot_t(oneh, out)


def _norm(z2, batch2d, segz, segz2, cnt, gw, gb, gm):
    return pl.pallas_call(
        _norm_body,
        grid=(NP // NB,),
        in_specs=[
            pl.BlockSpec((NB, D), lambda i: (i, 0)),
            pl.BlockSpec((NB, 1), lambda i: (i, 0)),
            pl.BlockSpec((G, D), lambda i: (0, 0)),
            pl.BlockSpec((G, D), lambda i: (0, 0)),
            pl.BlockSpec((G, 1), lambda i: (0, 0)),
            pl.BlockSpec((1, D), lambda i: (0, 0)),
            pl.BlockSpec((1, D), lambda i: (0, 0)),
            pl.BlockSpec((1, D), lambda i: (0, 0)),
        ],
        out_specs=[
            pl.BlockSpec((NB, D), lambda i: (i, 0)),
            pl.BlockSpec((G, D), lambda i: (0, 0)),
        ],
        out_shape=[
            jax.ShapeDtypeStruct((NP, D), jnp.float32),
            jax.ShapeDtypeStruct((G, D), jnp.float32),
        ],
    )(z2, batch2d, segz, segz2, cnt,
      gw.reshape(1, D), gb.reshape(1, D), gm.reshape(1, D))


def _final_body(pool_ref, cnt_ref, wl_ref, bl_ref, out_ref):
    cnt = jnp.maximum(cnt_ref[...], 1.0)
    pooled = pool_ref[...] / cnt
    out_ref[...] = _dot(pooled, wl_ref[...]) + bl_ref[...]


def _final(pool, cnt, Wl, bl):
    return pl.pallas_call(
        _final_body,
        out_shape=jax.ShapeDtypeStruct((G, D), jnp.float32),
    )(pool, cnt, Wl, bl.reshape(1, D))


def kernel(x, edge_index, edge_attr, batch,
           We0, be0, W10, b10, W20, b20, gw0, gb0, gm0,
           We1, be1, W11, b11, W21, b21, gw1, gb1, gm1, Wl, bl):
    src = edge_index[0].astype(jnp.int32)
    dst = edge_index[1].astype(jnp.int32)
    src_p = jnp.concatenate(
        [src, jnp.zeros((EP - E,), jnp.int32)]).reshape(NW, CPW, CH)
    dst_p = jnp.concatenate(
        [dst, jnp.full((EP - E,), TRASH, jnp.int32)]).reshape(NW, CPW, CH)
    ea_p = jnp.concatenate(
        [edge_attr, jnp.zeros((EP - E, 16), jnp.float32)], axis=0)
    x_p = jnp.concatenate([x, jnp.zeros((NP - N, D), jnp.float32)], axis=0)
    batch2d = jnp.concatenate(
        [batch.astype(jnp.int32), jnp.full((NP - N,), G, jnp.int32)]
    ).reshape(NP, 1)
    zeros_nd = jnp.zeros((NP, D), jnp.float32)

    eemb0 = _edge_embed(ea_p, We0, be0)
    eemb1 = _edge_embed(ea_p, We1, be1)

    agg0 = _sc_agg(src_p, dst_p, eemb0, x_p, zeros_nd)
    z2_0, segz0, segz20, cnt = _mlp_stats(x_p, agg0, batch2d, W10, b10, W20, b20)
    h1, _ = _norm(z2_0, batch2d, segz0, segz20, cnt, gw0, gb0, gm0)

    agg1 = _sc_agg(src_p, dst_p, eemb1, h1, zeros_nd)
    z2_1, segz1, segz21, _ = _mlp_stats(h1, agg1, batch2d, W11, b11, W21, b21)
    _, pool = _norm(z2_1, batch2d, segz1, segz21, cnt, gw1, gb1, gm1)

    return _final(pool, cnt, Wl, bl)


# speedup vs baseline: 3.8459x; 1.0344x over previous
import functools

import jax
import jax.numpy as jnp
from jax import lax
from jax.experimental import pallas as pl
from jax.experimental.pallas import tpu as pltpu
from jax.experimental.pallas import tpu_sc as plsc

N = 10000
E = 320000
D = 128
G = 128

NP = 10240
TRASH = N
NW = 32
CH = 80
NBUF = 2
CPW = ((E + NW * CH - 1) // (NW * CH) + 5) // 6 * 6
EP = NW * CH * CPW
NB = 2560
EB = 8064

_HI = jax.lax.Precision.HIGHEST


def _dot(a, b, precision=_HI):
    return jnp.dot(a, b, precision=precision,
                   preferred_element_type=jnp.float32)


def _dot_t(a, b):
    return lax.dot_general(a, b, (((0,), (0,)), ((), ())), precision=_HI,
                           preferred_element_type=jnp.float32)


def _edge_embed_body(ea_ref, we_ref, be_ref, out_ref):
    out_ref[...] = _dot(ea_ref[...], we_ref[...],
                        precision=jax.lax.Precision.DEFAULT) + be_ref[...]


def _edge_embed(ea_p, We, be):
    return pl.pallas_call(
        _edge_embed_body,
        grid=(EP // EB,),
        in_specs=[
            pl.BlockSpec((EB, 16), lambda i: (i, 0)),
            pl.BlockSpec((16, D), lambda i: (0, 0)),
            pl.BlockSpec((1, D), lambda i: (0, 0)),
        ],
        out_specs=pl.BlockSpec((EB, D), lambda i: (i, 0)),
        out_shape=jax.ShapeDtypeStruct((EP, D), jnp.float32),
    )(ea_p, We, be.reshape(1, D))


def _sc_agg_body(src_hbm, dst_hbm, eemb_hbm, h_hbm, zeros_hbm, out_hbm,
                 sidx0, sidx1, didx0, didx1, didx2, m_bufs, e_bufs, agg_sh,
                 isem, gsem, ssem):
    sidx = (sidx0, sidx1)
    didx = (didx0, didx1, didx2)
    c = lax.axis_index("c")
    s = lax.axis_index("s")
    wid = s * 2 + c
    rpt = NP // 16
    pltpu.sync_copy(zeros_hbm.at[pl.ds(s * rpt, rpt)],
                    agg_sh.at[pl.ds(s * rpt, rpt)])

    def issue_idx(k, b, d):
        pltpu.async_copy(src_hbm.at[wid, k], sidx[b], isem.at[b])
        pltpu.async_copy(dst_hbm.at[wid, k], didx[d], isem.at[b])

    def wait_idx(k, b, d):
        pltpu.make_async_copy(src_hbm.at[wid, k], sidx[b],
                              isem.at[b]).wait()
        pltpu.make_async_copy(dst_hbm.at[wid, k], didx[d],
                              isem.at[b]).wait()

    def issue_gather(k, b):
        base = (wid * CPW + k) * CH
        pltpu.async_copy(h_hbm.at[sidx[b]], m_bufs.at[b], gsem.at[b])
        pltpu.async_copy(eemb_hbm.at[pl.ds(base, CH)], e_bufs.at[b],
                         gsem.at[b])

    def wait_gather(k, b):
        base = (wid * CPW + k) * CH
        pltpu.make_async_copy(h_hbm.at[sidx[b]], m_bufs.at[b],
                              gsem.at[b]).wait()
        pltpu.make_async_copy(eemb_hbm.at[pl.ds(base, CH)], e_bufs.at[b],
                              gsem.at[b]).wait()

    def wait_scatter(b, d):
        pltpu.make_async_copy(m_bufs.at[b], agg_sh.at[didx[d]],
                              ssem.at[b]).wait()

    issue_idx(0, 0, 0)
    issue_idx(1, 1, 1)
    wait_idx(0, 0, 0)
    issue_gather(0, 0)
    plsc.subcore_barrier()

    @pl.loop(0, CPW, step=6)
    def _(i):
        for bb in range(6):
            k = i + bb
            b = bb % NBUF
            ob = 1 - b
            d = bb % 3
            d1 = (bb + 1) % 3
            d2 = (bb + 2) % 3

            @pl.when(k >= 1)
            def _():
                wait_scatter(ob, d2)

            @pl.when(k + 1 < CPW)
            def _():
                wait_idx(k + 1, ob, d1)
                issue_gather(k + 1, ob)

            wait_gather(k, b)
            m_v = m_bufs.at[b]
            e_v = e_bufs.at[b]

            @pl.loop(0, CH, step=2)
            def _(r):
                for rr in range(2):
                    for j in range(D // 16):
                        sl = (r + rr, pl.ds(j * 16, 16))
                        m_v.at[*sl][...] = jnp.maximum(
                            m_v.at[*sl][...] + e_v.at[*sl][...], 0.0)

            pltpu.async_copy(m_v, agg_sh.at[didx[d]], ssem.at[b], add=True)

            @pl.when(k + 2 < CPW)
            def _():
                issue_idx(k + 2, b, d2)

    wait_scatter((CPW - 1) % 2, (CPW - 1) % 3)
    plsc.subcore_barrier()
    pltpu.sync_copy(agg_sh.at[pl.ds(s * rpt, rpt)],
                    out_hbm.at[c, pl.ds(s * rpt, rpt)])


@functools.lru_cache(maxsize=1)
def _make_sc_agg():
    mesh = plsc.VectorSubcoreMesh(core_axis_name="c", subcore_axis_name="s",
                                  num_cores=2, num_subcores=16)
    return pl.kernel(
        _sc_agg_body,
        out_type=jax.ShapeDtypeStruct((2, NP, D), jnp.float32),
        mesh=mesh,
        scratch_types=[
            pltpu.VMEM((CH,), jnp.int32),
            pltpu.VMEM((CH,), jnp.int32),
            pltpu.VMEM((CH,), jnp.int32),
            pltpu.VMEM((CH,), jnp.int32),
            pltpu.VMEM((CH,), jnp.int32),
            pltpu.VMEM((NBUF, CH, D), jnp.float32),
            pltpu.VMEM((NBUF, CH, D), jnp.float32),
            pltpu.VMEM_SHARED((NP, D), jnp.float32),
            pltpu.SemaphoreType.DMA((NBUF,)),
            pltpu.SemaphoreType.DMA((NBUF,)),
            pltpu.SemaphoreType.DMA((NBUF,)),
        ],
    )


def _sc_agg(src2d, dst2d, eemb, h, zeros_nd):
    return _make_sc_agg()(src2d, dst2d, eemb, h, zeros_nd)


def _mlp_stats_body(h_ref, agg_ref, b_ref, w1_ref, b1_ref, w2_ref, b2_ref,
                    z2_ref, segz_ref, segz2_ref, cnt_ref):
    i = pl.program_id(0)
    z = h_ref[...] + agg_ref[0] + agg_ref[1]
    z1 = jnp.maximum(_dot(z, w1_ref[...]) + b1_ref[...], 0.0)
    z2 = _dot(z1, w2_ref[...]) + b2_ref[...]
    z2_ref[...] = z2
    gids = lax.broadcasted_iota(jnp.int32, (1, G), 1)
    oneh = (b_ref[...] == gids).astype(jnp.float32)

    @pl.when(i == 0)
    def _():
        segz_ref[...] = jnp.zeros_like(segz_ref)
        segz2_ref[...] = jnp.zeros_like(segz2_ref)
        cnt_ref[...] = jnp.zeros_like(cnt_ref)

    segz_ref[...] += _dot_t(oneh, z2)
    segz2_ref[...] += _dot_t(oneh, z2 * z2)
    cnt_ref[...] += _dot_t(oneh, jnp.ones((NB, 1), jnp.float32))


def _mlp_stats(h, aggpair, batch2d, W1, b1, W2, b2):
    return pl.pallas_call(
        _mlp_stats_body,
        grid=(NP // NB,),
        in_specs=[
            pl.BlockSpec((NB, D), lambda i: (i, 0)),
            pl.BlockSpec((2, NB, D), lambda i: (0, i, 0)),
            pl.BlockSpec((NB, 1), lambda i: (i, 0)),
            pl.BlockSpec((D, D), lambda i: (0, 0)),
            pl.BlockSpec((1, D), lambda i: (0, 0)),
            pl.BlockSpec((D, D), lambda i: (0, 0)),
            pl.BlockSpec((1, D), lambda i: (0, 0)),
        ],
        out_specs=[
            pl.BlockSpec((NB, D), lambda i: (i, 0)),
            pl.BlockSpec((G, D), lambda i: (0, 0)),
            pl.BlockSpec((G, D), lambda i: (0, 0)),
            pl.BlockSpec((G, 1), lambda i: (0, 0)),
        ],
        out_shape=[
            jax.ShapeDtypeStruct((NP, D), jnp.float32),
            jax.ShapeDtypeStruct((G, D), jnp.float32),
            jax.ShapeDtypeStruct((G, D), jnp.float32),
            jax.ShapeDtypeStruct((G, 1), jnp.float32),
        ],
    )(h, aggpair, batch2d, W1, b1.reshape(1, D), W2, b2.reshape(1, D))


def _norm_body(z2_ref, b_ref, segz_ref, segz2_ref, cnt_ref,
               gw_ref, gb_ref, gm_ref, out_ref, pool_ref):
    i = pl.program_id(0)
    cnt = jnp.maximum(cnt_ref[...], 1.0)
    mean = segz_ref[...] / cnt
    msq = segz2_ref[...] / cnt
    gm = gm_ref[...]
    var = msq - (2.0 - gm) * gm * mean * mean
    inv = gw_ref[...] * lax.rsqrt(var + 1e-5)
    alpha = inv
    beta = gb_ref[...] - inv * gm * mean
    gids = lax.broadcasted_iota(jnp.int32, (1, G), 1)
    oneh = (b_ref[...] == gids).astype(jnp.float32)
    a_b = _dot(oneh, alpha)
    b_b = _dot(oneh, beta)
    out = jnp.maximum(a_b * z2_ref[...] + b_b, 0.0)
    out_ref[...] = out

    @pl.when(i == 0)
    def _():
        pool_ref[...] = jnp.zeros_like(pool_ref)

    pool_ref[...] += _dot_t(oneh, out)


def _norm(z2, batch2d, segz, segz2, cnt, gw, gb, gm):
    return pl.pallas_call(
        _norm_body,
        grid=(NP // NB,),
        in_specs=[
            pl.BlockSpec((NB, D), lambda i: (i, 0)),
            pl.BlockSpec((NB, 1), lambda i: (i, 0)),
            pl.BlockSpec((G, D), lambda i: (0, 0)),
            pl.BlockSpec((G, D), lambda i: (0, 0)),
            pl.BlockSpec((G, 1), lambda i: (0, 0)),
            pl.BlockSpec((1, D), lambda i: (0, 0)),
            pl.BlockSpec((1, D), lambda i: (0, 0)),
            pl.BlockSpec((1, D), lambda i: (0, 0)),
        ],
        out_specs=[
            pl.BlockSpec((NB, D), lambda i: (i, 0)),
            pl.BlockSpec((G, D), lambda i: (0, 0)),
        ],
        out_shape=[
            jax.ShapeDtypeStruct((NP, D), jnp.float32),
            jax.ShapeDtypeStruct((G, D), jnp.float32),
        ],
    )(z2, batch2d, segz, segz2, cnt,
      gw.reshape(1, D), gb.reshape(1, D), gm.reshape(1, D))


def _final_body(pool_ref, cnt_ref, wl_ref, bl_ref, out_ref):
    cnt = jnp.maximum(cnt_ref[...], 1.0)
    pooled = pool_ref[...] / cnt
    out_ref[...] = _dot(pooled, wl_ref[...]) + bl_ref[...]


def _final(pool, cnt, Wl, bl):
    return pl.pallas_call(
        _final_body,
        out_shape=jax.ShapeDtypeStruct((G, D), jnp.float32),
    )(pool, cnt, Wl, bl.reshape(1, D))


def kernel(x, edge_index, edge_attr, batch,
           We0, be0, W10, b10, W20, b20, gw0, gb0, gm0,
           We1, be1, W11, b11, W21, b21, gw1, gb1, gm1, Wl, bl):
    src = edge_index[0].astype(jnp.int32)
    dst = edge_index[1].astype(jnp.int32)
    src_p = jnp.concatenate(
        [src, jnp.zeros((EP - E,), jnp.int32)]).reshape(NW, CPW, CH)
    dst_p = jnp.concatenate(
        [dst, jnp.full((EP - E,), TRASH, jnp.int32)]).reshape(NW, CPW, CH)
    ea_p = jnp.concatenate(
        [edge_attr, jnp.zeros((EP - E, 16), jnp.float32)], axis=0)
    x_p = jnp.concatenate([x, jnp.zeros((NP - N, D), jnp.float32)], axis=0)
    batch2d = jnp.concatenate(
        [batch.astype(jnp.int32), jnp.full((NP - N,), G, jnp.int32)]
    ).reshape(NP, 1)
    zeros_nd = jnp.zeros((NP, D), jnp.float32)

    eemb0 = _edge_embed(ea_p, We0, be0)
    eemb1 = _edge_embed(ea_p, We1, be1)

    agg0 = _sc_agg(src_p, dst_p, eemb0, x_p, zeros_nd)
    z2_0, segz0, segz20, cnt = _mlp_stats(x_p, agg0, batch2d, W10, b10, W20, b20)
    h1, _ = _norm(z2_0, batch2d, segz0, segz20, cnt, gw0, gb0, gm0)

    agg1 = _sc_agg(src_p, dst_p, eemb1, h1, zeros_nd)
    z2_1, segz1, segz21, _ = _mlp_stats(h1, agg1, batch2d, W11, b11, W21, b21)
    _, pool = _norm(z2_1, batch2d, segz1, segz21, cnt, gw1, gb1, gm1)

    return _final(pool, cnt, Wl, bl)
